# baseline (speedup 1.0000x reference)
import jax
import jax.numpy as jnp
from jax import lax
from jax.experimental import pallas as pl
from jax.experimental.pallas import tpu as pltpu
from jax.experimental.pallas import tpu_sc as plsc

N_RHO, N_THETA = 5, 8
OUT_DIM = 32
N_FEAT = 128
K_TOTAL = N_RHO * N_THETA
NP_ROWS = K_TOTAL * 3
X_W = K_TOTAL * N_FEAT
CHUNK = 20
NC, NS = 2, 16
NW = NC * NS
LANES = 16
VREGS_PER_ROW = N_FEAT // LANES


def _pullback_vertex(vl, rows, wgt_v, xv):
    wbase = vl * NP_ROWS

    def i_body(i, carry):
        base24 = wbase + i * (N_THETA * 3)
        w_lo = wgt_v[pl.ds(base24, LANES)]
        w_hi = wgt_v[pl.ds(base24 + 8, LANES)]
        for j in range(N_THETA):
            q0 = 3 * j
            p = i * (N_THETA * 3) + q0
            t = i * N_THETA + j
            ws = [w_lo[q] if q < LANES else w_hi[q - 8]
                  for q in (q0, q0 + 1, q0 + 2)]
            for h in range(VREGS_PER_ROW):
                sl = pl.ds(h * LANES, LANES)
                g0 = rows[p, sl]
                g1 = rows[p + 1, sl]
                g2 = rows[p + 2, sl]
                xv[pl.ds(t * N_FEAT + h * LANES, LANES)] = (
                    ws[0] * g0 + ws[1] * g1) + ws[2] * g2
        return carry

    lax.fori_loop(0, N_RHO, i_body, 0)


def _make_sc_kernel(m_pad):
    v_per_w = m_pad // NW
    n_chunks = v_per_w // CHUNK

    def body(sig_hbm, gidx_hbm, wgt_hbm, x_hbm,
             idx_v, wgt_v, rows0, rows1, xv0, xv1,
             gsem0, gsem1, osem0, osem1):
        wid = lax.axis_index("s") * NC + lax.axis_index("c")

        def fire(vl, rows, sem):
            pltpu.async_copy(
                sig_hbm.at[idx_v.at[pl.ds(vl * NP_ROWS, NP_ROWS)]], rows, sem)

        def drain(vl, rows, sem):
            pltpu.make_async_copy(
                sig_hbm.at[idx_v.at[pl.ds(vl * NP_ROWS, NP_ROWS)]], rows,
                sem).wait()

        def xslice(vg):
            return x_hbm.at[pl.ds(vg * X_W, X_W)]

        def chunk_body(c, _):
            vbase = wid * v_per_w + c * CHUNK
            pltpu.sync_copy(
                gidx_hbm.at[pl.ds(vbase * NP_ROWS, CHUNK * NP_ROWS)], idx_v)
            pltpu.sync_copy(
                wgt_hbm.at[pl.ds(vbase * NP_ROWS, CHUNK * NP_ROWS)], wgt_v)
            fire(0, rows0, gsem0)
            fire(1, rows1, gsem1)

            def pair_body(vp, _):
                for bi, (rows, gsem, xv, osem) in enumerate(
                        ((rows0, gsem0, xv0, osem0),
                         (rows1, gsem1, xv1, osem1))):
                    vl = 2 * vp + bi
                    drain(vl, rows, gsem)

                    @pl.when(vl >= 2)
                    def _():
                        pltpu.make_async_copy(
                            xv, xslice(vbase + vl - 2), osem).wait()

                    _pullback_vertex(vl, rows, wgt_v, xv)
                    pltpu.async_copy(xv, xslice(vbase + vl), osem)
                    nxt = vl + 2

                    @pl.when(nxt < CHUNK)
                    def _():
                        fire(nxt, rows, gsem)
                return 0

            lax.fori_loop(0, CHUNK // 2, pair_body, 0)
            pltpu.make_async_copy(
                xv0, xslice(vbase + CHUNK - 2), osem0).wait()
            pltpu.make_async_copy(
                xv1, xslice(vbase + CHUNK - 1), osem1).wait()
            return 0

        lax.fori_loop(0, n_chunks, chunk_body, 0)

    mesh = plsc.VectorSubcoreMesh(core_axis_name="c", subcore_axis_name="s")
    return pl.kernel(
        body,
        out_type=jax.ShapeDtypeStruct((m_pad * X_W,), jnp.float32),
        mesh=mesh,
        compiler_params=pltpu.CompilerParams(needs_layout_passes=False),
        scratch_types=[
            pltpu.VMEM((CHUNK * NP_ROWS,), jnp.int32),
            pltpu.VMEM((CHUNK * NP_ROWS,), jnp.float32),
            pltpu.VMEM((NP_ROWS, N_FEAT), jnp.float32),
            pltpu.VMEM((NP_ROWS, N_FEAT), jnp.float32),
            pltpu.VMEM((X_W,), jnp.float32),
            pltpu.VMEM((X_W,), jnp.float32),
            pltpu.SemaphoreType.DMA,
            pltpu.SemaphoreType.DMA,
            pltpu.SemaphoreType.DMA,
            pltpu.SemaphoreType.DMA,
        ],
    )


def _conv_body(x_ref, k_ref, o_ref):
    xb = x_ref[...]
    best = None
    best_n = None
    for r in range(N_THETA):
        s = lax.dot_general(
            xb, k_ref[r], (((1,), (0,)), ((), ())),
            preferred_element_type=jnp.float32)
        s = jnp.maximum(s, 0.0)
        n = jnp.sum(s * s, axis=1)
        if r == 0:
            best, best_n = s, n
        else:
            take = n > best_n
            best = jnp.where(take[:, None], s, best)
            best_n = jnp.where(take, n, best_n)
    o_ref[...] = best


def _conv_pool(x, kmats):
    m = x.shape[0]
    bm = 320
    assert m % bm == 0
    return pl.pallas_call(
        _conv_body,
        grid=(m // bm,),
        in_specs=[
            pl.BlockSpec((bm, X_W), lambda i: (i, 0)),
            pl.BlockSpec((N_THETA, X_W, OUT_DIM), lambda i: (0, 0, 0)),
        ],
        out_specs=pl.BlockSpec((bm, OUT_DIM), lambda i: (i, 0)),
        out_shape=jax.ShapeDtypeStruct((m, OUT_DIM), jnp.float32),
    )(x, kmats)


def kernel(signal, bary_coords, kernel):
    b, m, _ = signal.shape
    krot = jnp.stack(
        [jnp.roll(kernel[0], r, axis=0) for r in range(N_THETA)])
    kmats = jnp.transpose(krot, (0, 2, 1, 4, 3)).reshape(
        N_THETA, X_W, OUT_DIM)
    outs = []
    for bi in range(b):
        sig = signal[bi]
        bary = bary_coords[bi]
        gidx = bary[:, :, 1::2][:, :, :3].astype(jnp.int32).reshape(
            m, NP_ROWS)
        wgt = bary[:, :, 0::2][:, :, :3].reshape(m, NP_ROWS)
        grain = NW * CHUNK
        m_pad = ((m + grain - 1) // grain) * grain
        gidx = jnp.pad(gidx, ((0, m_pad - m), (0, 0)), mode="wrap"
                       ).reshape(-1)
        wgt = jnp.pad(wgt, ((0, m_pad - m), (0, 0))).reshape(-1)
        nh = 16
        part = m_pad // nh
        pw = part * NP_ROWS
        sck = _make_sc_kernel(part)
        xs = [sck(sig, gidx[i * pw:(i + 1) * pw],
                  wgt[i * pw:(i + 1) * pw]).reshape(part, X_W)
              for i in range(nh)]
        os_ = [_conv_pool(xp, kmats) for xp in xs]
        outs.append(jnp.concatenate(os_, axis=0)[:m])
    return jnp.stack(outs, axis=0)


# speedup vs baseline: 1.0034x; 1.0034x over previous
import jax
import jax.numpy as jnp
from jax import lax
from jax.experimental import pallas as pl
from jax.experimental.pallas import tpu as pltpu
from jax.experimental.pallas import tpu_sc as plsc

N_RHO, N_THETA = 5, 8
OUT_DIM = 32
N_FEAT = 128
K_TOTAL = N_RHO * N_THETA
NP_ROWS = K_TOTAL * 3
X_W = K_TOTAL * N_FEAT
CHUNK = 20
NC, NS = 2, 16
NW = NC * NS
LANES = 16
VREGS_PER_ROW = N_FEAT // LANES


def _pullback_vertex(vl, rows, wgt_v, xv):
    wbase = vl * NP_ROWS

    def i_body(i, carry):
        base24 = wbase + i * (N_THETA * 3)
        w_lo = wgt_v[pl.ds(base24, LANES)]
        w_hi = wgt_v[pl.ds(base24 + 8, LANES)]
        for j in range(N_THETA):
            q0 = 3 * j
            p = i * (N_THETA * 3) + q0
            t = i * N_THETA + j
            ws = [w_lo[q] if q < LANES else w_hi[q - 8]
                  for q in (q0, q0 + 1, q0 + 2)]
            for h in range(VREGS_PER_ROW):
                sl = pl.ds(h * LANES, LANES)
                g0 = rows[p, sl]
                g1 = rows[p + 1, sl]
                g2 = rows[p + 2, sl]
                xv[pl.ds(t * N_FEAT + h * LANES, LANES)] = (
                    ws[0] * g0 + ws[1] * g1) + ws[2] * g2
        return carry

    lax.fori_loop(0, N_RHO, i_body, 0)


def _make_sc_kernel(m_pad):
    v_per_w = m_pad // NW
    n_chunks = v_per_w // CHUNK

    def body(sig_hbm, gidx_hbm, wgt_hbm, x_hbm,
             idx_v, wgt_v, rows0, rows1, xv0, xv1,
             gsem0, gsem1, osem0, osem1):
        wid = lax.axis_index("s") * NC + lax.axis_index("c")

        def fire(vl, rows, sem):
            pltpu.async_copy(
                sig_hbm.at[idx_v.at[pl.ds(vl * NP_ROWS, NP_ROWS)]], rows, sem)

        def drain(vl, rows, sem):
            pltpu.make_async_copy(
                sig_hbm.at[idx_v.at[pl.ds(vl * NP_ROWS, NP_ROWS)]], rows,
                sem).wait()

        def xslice(vg):
            return x_hbm.at[pl.ds(vg * X_W, X_W)]

        def chunk_body(c, _):
            vbase = wid * v_per_w + c * CHUNK
            pltpu.sync_copy(
                gidx_hbm.at[pl.ds(vbase * NP_ROWS, CHUNK * NP_ROWS)], idx_v)
            pltpu.sync_copy(
                wgt_hbm.at[pl.ds(vbase * NP_ROWS, CHUNK * NP_ROWS)], wgt_v)
            fire(0, rows0, gsem0)
            fire(1, rows1, gsem1)

            def pair_body(vp, _):
                for bi, (rows, gsem, xv, osem) in enumerate(
                        ((rows0, gsem0, xv0, osem0),
                         (rows1, gsem1, xv1, osem1))):
                    vl = 2 * vp + bi
                    drain(vl, rows, gsem)

                    @pl.when(vl >= 2)
                    def _():
                        pltpu.make_async_copy(
                            xv, xslice(vbase + vl - 2), osem).wait()

                    _pullback_vertex(vl, rows, wgt_v, xv)
                    pltpu.async_copy(xv, xslice(vbase + vl), osem)
                    nxt = vl + 2

                    @pl.when(nxt < CHUNK)
                    def _():
                        fire(nxt, rows, gsem)
                return 0

            lax.fori_loop(0, CHUNK // 2, pair_body, 0)
            pltpu.make_async_copy(
                xv0, xslice(vbase + CHUNK - 2), osem0).wait()
            pltpu.make_async_copy(
                xv1, xslice(vbase + CHUNK - 1), osem1).wait()
            return 0

        lax.fori_loop(0, n_chunks, chunk_body, 0)

    mesh = plsc.VectorSubcoreMesh(core_axis_name="c", subcore_axis_name="s")
    return pl.kernel(
        body,
        out_type=jax.ShapeDtypeStruct((m_pad * X_W,), jnp.float32),
        mesh=mesh,
        compiler_params=pltpu.CompilerParams(needs_layout_passes=False),
        scratch_types=[
            pltpu.VMEM((CHUNK * NP_ROWS,), jnp.int32),
            pltpu.VMEM((CHUNK * NP_ROWS,), jnp.float32),
            pltpu.VMEM((NP_ROWS, N_FEAT), jnp.float32),
            pltpu.VMEM((NP_ROWS, N_FEAT), jnp.float32),
            pltpu.VMEM((X_W,), jnp.float32),
            pltpu.VMEM((X_W,), jnp.float32),
            pltpu.SemaphoreType.DMA,
            pltpu.SemaphoreType.DMA,
            pltpu.SemaphoreType.DMA,
            pltpu.SemaphoreType.DMA,
        ],
    )


def _conv_body(x_ref, k_ref, o_ref):
    xb = x_ref[...]
    best = None
    best_n = None
    for r in range(N_THETA):
        s = lax.dot_general(
            xb, k_ref[r], (((1,), (0,)), ((), ())),
            preferred_element_type=jnp.float32)
        s = jnp.maximum(s, 0.0)
        n = jnp.sum(s * s, axis=1)
        if r == 0:
            best, best_n = s, n
        else:
            take = n > best_n
            best = jnp.where(take[:, None], s, best)
            best_n = jnp.where(take, n, best_n)
    o_ref[...] = best


def _conv_pool(x, kmats):
    m = x.shape[0]
    bm = 320
    assert m % bm == 0
    return pl.pallas_call(
        _conv_body,
        grid=(m // bm,),
        in_specs=[
            pl.BlockSpec((bm, X_W), lambda i: (i, 0)),
            pl.BlockSpec((N_THETA, X_W, OUT_DIM), lambda i: (0, 0, 0)),
        ],
        out_specs=pl.BlockSpec((bm, OUT_DIM), lambda i: (i, 0)),
        out_shape=jax.ShapeDtypeStruct((m, OUT_DIM), jnp.float32),
    )(x, kmats)


def kernel(signal, bary_coords, kernel):
    b, m, _ = signal.shape
    krot = jnp.stack(
        [jnp.roll(kernel[0], r, axis=0) for r in range(N_THETA)])
    kmats = jnp.transpose(krot, (0, 2, 1, 4, 3)).reshape(
        N_THETA, X_W, OUT_DIM)
    outs = []
    for bi in range(b):
        sig = signal[bi]
        bary = bary_coords[bi]
        gidx = bary[:, :, 1::2][:, :, :3].astype(jnp.int32).reshape(
            m, NP_ROWS)
        wgt = bary[:, :, 0::2][:, :, :3].reshape(m, NP_ROWS)
        grain = NW * CHUNK
        m_pad = ((m + grain - 1) // grain) * grain
        gidx = jnp.pad(gidx, ((0, m_pad - m), (0, 0)), mode="wrap"
                       ).reshape(-1)
        wgt = jnp.pad(wgt, ((0, m_pad - m), (0, 0))).reshape(-1)
        nh = 8
        part = m_pad // nh
        pw = part * NP_ROWS
        sck = _make_sc_kernel(part)
        xs = [sck(sig, gidx[i * pw:(i + 1) * pw],
                  wgt[i * pw:(i + 1) * pw]).reshape(part, X_W)
              for i in range(nh)]
        os_ = [_conv_pool(xp, kmats) for xp in xs]
        outs.append(jnp.concatenate(os_, axis=0)[:m])
    return jnp.stack(outs, axis=0)


# speedup vs baseline: 1.0336x; 1.0301x over previous
import jax
import jax.numpy as jnp
from jax import lax
from jax.experimental import pallas as pl
from jax.experimental.pallas import tpu as pltpu
from jax.experimental.pallas import tpu_sc as plsc

N_RHO, N_THETA = 5, 8
OUT_DIM = 32
N_FEAT = 128
K_TOTAL = N_RHO * N_THETA
NP_ROWS = K_TOTAL * 3
X_W = K_TOTAL * N_FEAT
CHUNK = 40
NC, NS = 2, 16
NW = NC * NS
LANES = 16
VREGS_PER_ROW = N_FEAT // LANES


def _pullback_vertex(vl, rows, wgt_v, xv):
    wbase = vl * NP_ROWS

    def i_body(i, carry):
        base24 = wbase + i * (N_THETA * 3)
        w_lo = wgt_v[pl.ds(base24, LANES)]
        w_hi = wgt_v[pl.ds(base24 + 8, LANES)]
        for j in range(N_THETA):
            q0 = 3 * j
            p = i * (N_THETA * 3) + q0
            t = i * N_THETA + j
            ws = [w_lo[q] if q < LANES else w_hi[q - 8]
                  for q in (q0, q0 + 1, q0 + 2)]
            for h in range(VREGS_PER_ROW):
                sl = pl.ds(h * LANES, LANES)
                g0 = rows[p, sl]
                g1 = rows[p + 1, sl]
                g2 = rows[p + 2, sl]
                xv[pl.ds(t * N_FEAT + h * LANES, LANES)] = (
                    ws[0] * g0 + ws[1] * g1) + ws[2] * g2
        return carry

    lax.fori_loop(0, N_RHO, i_body, 0)


def _make_sc_kernel(m_pad):
    v_per_w = m_pad // NW
    n_chunks = v_per_w // CHUNK

    def body(sig_hbm, gidx_hbm, wgt_hbm, x_hbm,
             idx_v, wgt_v, rows0, rows1, xv0, xv1,
             gsem0, gsem1, osem0, osem1):
        wid = lax.axis_index("s") * NC + lax.axis_index("c")

        def fire(vl, rows, sem):
            pltpu.async_copy(
                sig_hbm.at[idx_v.at[pl.ds(vl * NP_ROWS, NP_ROWS)]], rows, sem)

        def drain(vl, rows, sem):
            pltpu.make_async_copy(
                sig_hbm.at[idx_v.at[pl.ds(vl * NP_ROWS, NP_ROWS)]], rows,
                sem).wait()

        def xslice(vg):
            return x_hbm.at[pl.ds(vg * X_W, X_W)]

        def chunk_body(c, _):
            vbase = wid * v_per_w + c * CHUNK
            pltpu.sync_copy(
                gidx_hbm.at[pl.ds(vbase * NP_ROWS, CHUNK * NP_ROWS)], idx_v)
            pltpu.sync_copy(
                wgt_hbm.at[pl.ds(vbase * NP_ROWS, CHUNK * NP_ROWS)], wgt_v)
            fire(0, rows0, gsem0)
            fire(1, rows1, gsem1)

            def pair_body(vp, _):
                for bi, (rows, gsem, xv, osem) in enumerate(
                        ((rows0, gsem0, xv0, osem0),
                         (rows1, gsem1, xv1, osem1))):
                    vl = 2 * vp + bi
                    drain(vl, rows, gsem)

                    @pl.when(vl >= 2)
                    def _():
                        pltpu.make_async_copy(
                            xv, xslice(vbase + vl - 2), osem).wait()

                    _pullback_vertex(vl, rows, wgt_v, xv)
                    pltpu.async_copy(xv, xslice(vbase + vl), osem)
                    nxt = vl + 2

                    @pl.when(nxt < CHUNK)
                    def _():
                        fire(nxt, rows, gsem)
                return 0

            lax.fori_loop(0, CHUNK // 2, pair_body, 0)
            pltpu.make_async_copy(
                xv0, xslice(vbase + CHUNK - 2), osem0).wait()
            pltpu.make_async_copy(
                xv1, xslice(vbase + CHUNK - 1), osem1).wait()
            return 0

        lax.fori_loop(0, n_chunks, chunk_body, 0)

    mesh = plsc.VectorSubcoreMesh(core_axis_name="c", subcore_axis_name="s")
    return pl.kernel(
        body,
        out_type=jax.ShapeDtypeStruct((m_pad * X_W,), jnp.float32),
        mesh=mesh,
        compiler_params=pltpu.CompilerParams(needs_layout_passes=False),
        scratch_types=[
            pltpu.VMEM((CHUNK * NP_ROWS,), jnp.int32),
            pltpu.VMEM((CHUNK * NP_ROWS,), jnp.float32),
            pltpu.VMEM((NP_ROWS, N_FEAT), jnp.float32),
            pltpu.VMEM((NP_ROWS, N_FEAT), jnp.float32),
            pltpu.VMEM((X_W,), jnp.float32),
            pltpu.VMEM((X_W,), jnp.float32),
            pltpu.SemaphoreType.DMA,
            pltpu.SemaphoreType.DMA,
            pltpu.SemaphoreType.DMA,
            pltpu.SemaphoreType.DMA,
        ],
    )


def _conv_body(x_ref, k_ref, o_ref):
    xb = x_ref[...]
    best = None
    best_n = None
    for r in range(N_THETA):
        s = lax.dot_general(
            xb, k_ref[r], (((1,), (0,)), ((), ())),
            preferred_element_type=jnp.float32)
        s = jnp.maximum(s, 0.0)
        n = jnp.sum(s * s, axis=1)
        if r == 0:
            best, best_n = s, n
        else:
            take = n > best_n
            best = jnp.where(take[:, None], s, best)
            best_n = jnp.where(take, n, best_n)
    o_ref[...] = best


def _conv_pool(x, kmats):
    m = x.shape[0]
    bm = 320
    assert m % bm == 0
    return pl.pallas_call(
        _conv_body,
        grid=(m // bm,),
        in_specs=[
            pl.BlockSpec((bm, X_W), lambda i: (i, 0)),
            pl.BlockSpec((N_THETA, X_W, OUT_DIM), lambda i: (0, 0, 0)),
        ],
        out_specs=pl.BlockSpec((bm, OUT_DIM), lambda i: (i, 0)),
        out_shape=jax.ShapeDtypeStruct((m, OUT_DIM), jnp.float32),
    )(x, kmats)


def kernel(signal, bary_coords, kernel):
    b, m, _ = signal.shape
    krot = jnp.stack(
        [jnp.roll(kernel[0], r, axis=0) for r in range(N_THETA)])
    kmats = jnp.transpose(krot, (0, 2, 1, 4, 3)).reshape(
        N_THETA, X_W, OUT_DIM)
    outs = []
    for bi in range(b):
        sig = signal[bi]
        bary = bary_coords[bi]
        gidx = bary[:, :, 1::2][:, :, :3].astype(jnp.int32).reshape(
            m, NP_ROWS)
        wgt = bary[:, :, 0::2][:, :, :3].reshape(m, NP_ROWS)
        grain = NW * CHUNK
        m_pad = ((m + grain - 1) // grain) * grain
        gidx = jnp.pad(gidx, ((0, m_pad - m), (0, 0)), mode="wrap"
                       ).reshape(-1)
        wgt = jnp.pad(wgt, ((0, m_pad - m), (0, 0))).reshape(-1)
        nh = 8
        part = m_pad // nh
        pw = part * NP_ROWS
        sck = _make_sc_kernel(part)
        xs = [sck(sig, gidx[i * pw:(i + 1) * pw],
                  wgt[i * pw:(i + 1) * pw]).reshape(part, X_W)
              for i in range(nh)]
        os_ = [_conv_pool(xp, kmats) for xp in xs]
        outs.append(jnp.concatenate(os_, axis=0)[:m])
    return jnp.stack(outs, axis=0)
